# baseline (speedup 1.0000x reference)
import functools

import jax
import jax.numpy as jnp
from jax import lax
from jax.experimental import pallas as pl
from jax.experimental.pallas import tpu as pltpu
from jax.experimental.pallas import tpu_sc as plsc

N_ENT = 1000000
N_REL = 1000
DIM = 32
BATCH = 16384
L = 16

NUM_CORES = 2
NUM_SUBCORES = 16
NW = NUM_CORES * NUM_SUBCORES
BPW = BATCH // NW
CHUNK = 128
NCHUNK = BPW // CHUNK
GROUPS = CHUNK // L
ACC_STRIDE = L + 1


def _body(h_hbm, t_hbm, nh_hbm, nt_hbm, r_hbm,
          sc_ent, re_ent, im_ent, sc_rel, re_rel, im_rel,
          pos_hbm, neg_hbm,
          ih, it, inh, int_, ir,
          b_sc_h, b_re_h, b_im_h, b_sc_t, b_re_t, b_im_t,
          b_sc_nh, b_re_nh, b_im_nh, b_sc_nt, b_re_nt, b_im_nt,
          b_sc_r, b_re_r, b_im_r,
          acc_p, acc_n, out_p, out_n, sem):
    wid = lax.axis_index("s") * NUM_CORES + lax.axis_index("c")
    base = wid * BPW

    pltpu.sync_copy(h_hbm.at[pl.ds(base, BPW)], ih)
    pltpu.sync_copy(t_hbm.at[pl.ds(base, BPW)], it)
    pltpu.sync_copy(nh_hbm.at[pl.ds(base, BPW)], inh)
    pltpu.sync_copy(nt_hbm.at[pl.ds(base, BPW)], int_)
    pltpu.sync_copy(r_hbm.at[pl.ds(base, BPW)], ir)

    iota = lax.iota(jnp.int32, L)
    zero = jnp.zeros((L,), jnp.float32)

    gathers = (
        (sc_ent, ih, b_sc_h), (re_ent, ih, b_re_h), (im_ent, ih, b_im_h),
        (sc_ent, it, b_sc_t), (re_ent, it, b_re_t), (im_ent, it, b_im_t),
        (sc_ent, inh, b_sc_nh), (re_ent, inh, b_re_nh), (im_ent, inh, b_im_nh),
        (sc_ent, int_, b_sc_nt), (re_ent, int_, b_re_nt), (im_ent, int_, b_im_nt),
        (sc_rel, ir, b_sc_r), (re_rel, ir, b_re_r), (im_rel, ir, b_im_r),
    )

    for c in range(NCHUNK):
        cps = [pltpu.async_copy(tbl.at[idx.at[pl.ds(c * CHUNK, CHUNK)]], dst, sem)
               for (tbl, idx, dst) in gathers]
        for cp in cps:
            cp.wait()

        def group(g, _):
            for e in range(L):
                b = g * L + e
                acc_pos = zero
                acc_neg = zero
                for half in (0, L):
                    s = pl.ds(half, L)
                    scr = b_sc_r[b, s]
                    rer = b_re_r[b, s]
                    imr = b_im_r[b, s]
                    sch = b_sc_h[b, s]
                    reh = b_re_h[b, s]
                    imh = b_im_h[b, s]
                    sct = b_sc_t[b, s]
                    ret = b_re_t[b, s]
                    imt = b_im_t[b, s]
                    acc_pos = (acc_pos + sch * scr * sct
                               + reh * (rer * ret + imr * imt)
                               + imh * (rer * imt - imr * ret))
                    scnh = b_sc_nh[b, s]
                    renh = b_re_nh[b, s]
                    imnh = b_im_nh[b, s]
                    scnt = b_sc_nt[b, s]
                    rent = b_re_nt[b, s]
                    imnt = b_im_nt[b, s]
                    acc_neg = (acc_neg + scnh * scr * scnt
                               + renh * (rer * rent + imr * imnt)
                               + imnh * (rer * imnt - imr * rent))
                plsc.store_scatter(acc_p, [iota + e * ACC_STRIDE], acc_pos)
                plsc.store_scatter(acc_n, [iota + e * ACC_STRIDE], acc_neg)
            ssum_p = zero
            ssum_n = zero
            cols = iota * ACC_STRIDE
            for d in range(L):
                ssum_p = ssum_p + plsc.load_gather(acc_p, [cols + d])
                ssum_n = ssum_n + plsc.load_gather(acc_n, [cols + d])
            o = c * CHUNK + g * L
            plsc.store_scatter(out_p, [iota + o], ssum_p)
            plsc.store_scatter(out_n, [iota + o], ssum_n)
            return 0

        lax.fori_loop(0, GROUPS, group, 0)

    pltpu.sync_copy(out_p, pos_hbm.at[pl.ds(base, BPW)])
    pltpu.sync_copy(out_n, neg_hbm.at[pl.ds(base, BPW)])


@jax.jit
def kernel(h, t, nh, nt, r, sc_ent, re_ent, im_ent, sc_rel, re_rel, im_rel):
    mesh = plsc.VectorSubcoreMesh(
        core_axis_name="c", subcore_axis_name="s",
        num_cores=NUM_CORES, num_subcores=NUM_SUBCORES)
    run = pl.kernel(
        _body,
        out_type=(jax.ShapeDtypeStruct((BATCH,), jnp.float32),
                  jax.ShapeDtypeStruct((BATCH,), jnp.float32)),
        mesh=mesh,
        compiler_params=pltpu.CompilerParams(
            needs_layout_passes=False, use_tc_tiling_on_sc=False),
        scratch_types=(
            [pltpu.VMEM((BPW,), jnp.int32)] * 5
            + [pltpu.VMEM((CHUNK, DIM), jnp.float32)] * 15
            + [pltpu.VMEM((L * ACC_STRIDE,), jnp.float32)] * 2
            + [pltpu.VMEM((BPW,), jnp.float32)] * 2
            + [pltpu.SemaphoreType.DMA]
        ),
    )
    return run(h, t, nh, nt, r, sc_ent, re_ent, im_ent, sc_rel, re_rel, im_rel)
